# baseline (speedup 1.0000x reference)
import functools

import jax
import jax.numpy as jnp
from jax import lax
from jax.experimental import pallas as pl
from jax.experimental.pallas import tpu as pltpu
from jax.experimental.pallas import tpu_sc as plsc

VOCAB = 1000000
SEQ = 200
EMBED = 64
BATCH = 4096
ROWS = BATCH * SEQ
NC, NS = 2, 16
NW = NC * NS
RPW = ROWS // NW
SEQ_PER_W = RPW // SEQ
HALF = 100
IDX_ROWS = RPW // HALF


def _body(tok_hbm, table_hbm, pos_hbm, out_hbm, idx_v, pos_v, rows_v, sem):
    wid = lax.axis_index("s") * NC + lax.axis_index("c")
    base_idx_row = wid * IDX_ROWS
    pltpu.sync_copy(tok_hbm.at[pl.ds(base_idx_row, IDX_ROWS)], idx_v)
    pltpu.sync_copy(pos_hbm, pos_v)

    def seq_body(s, carry):
        h0 = pltpu.async_copy(
            table_hbm.at[idx_v.at[2 * s]], rows_v.at[pl.ds(0, HALF)], sem)
        h1 = pltpu.async_copy(
            table_hbm.at[idx_v.at[2 * s + 1]], rows_v.at[pl.ds(HALF, HALF)], sem)
        h0.wait()
        h1.wait()

        def add_row(r, c2):
            for c in range(EMBED // 16):
                sl = pl.ds(c * 16, 16)
                plsc.addupdate(rows_v.at[r, sl], pos_v[r, sl])
            return c2

        lax.fori_loop(0, SEQ, add_row, 0)
        out_base = wid * RPW + s * SEQ
        pltpu.sync_copy(rows_v, out_hbm.at[pl.ds(out_base, SEQ)])
        return carry

    lax.fori_loop(0, SEQ_PER_W, seq_body, 0)


_grid_kernel = pl.kernel(
    _body,
    out_type=jax.ShapeDtypeStruct((ROWS, EMBED), jnp.float32),
    mesh=plsc.VectorSubcoreMesh(core_axis_name="c", subcore_axis_name="s"),
    scratch_types=[
        pltpu.VMEM((IDX_ROWS, HALF), jnp.int32),
        pltpu.VMEM((SEQ, EMBED), jnp.float32),
        pltpu.VMEM((SEQ, EMBED), jnp.float32),
        pltpu.SemaphoreType.DMA,
    ],
    compiler_params=pltpu.CompilerParams(use_tc_tiling_on_sc=False),
)


@jax.jit
def kernel(tokens, word_table, pos_table):
    tok2 = tokens.astype(jnp.int32).reshape(ROWS // HALF, HALF)
    out = _grid_kernel(tok2, word_table, pos_table)
    return out.reshape(BATCH, SEQ, EMBED)


# speedup vs baseline: 1.1525x; 1.1525x over previous
import jax
import jax.numpy as jnp
from jax import lax
from jax.experimental import pallas as pl
from jax.experimental.pallas import tpu as pltpu
from jax.experimental.pallas import tpu_sc as plsc

VOCAB = 1000000
SEQ = 200
EMBED = 64
BATCH = 4096
ROWS = BATCH * SEQ
NC, NS = 2, 16
NW = NC * NS
RPW = ROWS // NW
SEQ_PER_W = RPW // SEQ
HALF = 100
IDX_ROWS = RPW // HALF
BLK = 2
BLK_ROWS = BLK * SEQ
NBLK = SEQ_PER_W // BLK
CHUNKS = BLK_ROWS // HALF


def _body(tok_hbm, table_hbm, pos_hbm, out_hbm,
          idx_v, pos_v, buf0, buf1, sg0, sg1, ss0, ss1):
    wid = lax.axis_index("s") * NC + lax.axis_index("c")
    pltpu.sync_copy(tok_hbm.at[pl.ds(wid * IDX_ROWS, IDX_ROWS)], idx_v)
    pltpu.sync_copy(pos_hbm, pos_v)
    bufs = (buf0, buf1)
    sgs = (sg0, sg1)
    sss = (ss0, ss1)
    out_w = wid * RPW

    def fire_gathers(k, pp):
        for j in range(CHUNKS):
            pltpu.async_copy(table_hbm.at[idx_v.at[CHUNKS * k + j]],
                             bufs[pp].at[pl.ds(j * HALF, HALF)], sgs[pp])

    def wait_gathers(k, pp):
        for j in range(CHUNKS):
            pltpu.make_async_copy(table_hbm.at[idx_v.at[CHUNKS * k + j]],
                                  bufs[pp].at[pl.ds(j * HALF, HALF)],
                                  sgs[pp]).wait()

    def fire_store(k, pp):
        pltpu.async_copy(bufs[pp],
                         out_hbm.at[pl.ds(out_w + k * BLK_ROWS, BLK_ROWS)],
                         sss[pp])

    def wait_store(k, pp):
        pltpu.make_async_copy(bufs[pp],
                              out_hbm.at[pl.ds(out_w + k * BLK_ROWS, BLK_ROWS)],
                              sss[pp]).wait()

    def add_pos(pp):
        buf = bufs[pp]

        def add_row(r, carry):
            for c in range(EMBED // 16):
                sl = pl.ds(c * 16, 16)
                p = pos_v[r, sl]
                plsc.addupdate(buf.at[r, sl], p)
                plsc.addupdate(buf.at[r + SEQ, sl], p)
            return carry

        lax.fori_loop(0, SEQ, add_row, 0, unroll=2)

    fire_gathers(0, 0)

    def outer(k2, carry):
        for b in range(2):
            k = 2 * k2 + b
            pp = b

            @pl.when(k + 1 < NBLK)
            def _():
                @pl.when(k >= 1)
                def _():
                    wait_store(k - 1, 1 - pp)
                fire_gathers(k + 1, 1 - pp)

            wait_gathers(k, pp)
            add_pos(pp)
            fire_store(k, pp)
        return carry

    lax.fori_loop(0, NBLK // 2, outer, 0)
    wait_store(NBLK - 2, 0)
    wait_store(NBLK - 1, 1)


_grid_kernel = pl.kernel(
    _body,
    out_type=jax.ShapeDtypeStruct((ROWS, EMBED), jnp.float32),
    mesh=plsc.VectorSubcoreMesh(core_axis_name="c", subcore_axis_name="s"),
    scratch_types=[
        pltpu.VMEM((IDX_ROWS, HALF), jnp.int32),
        pltpu.VMEM((SEQ, EMBED), jnp.float32),
        pltpu.VMEM((BLK_ROWS, EMBED), jnp.float32),
        pltpu.VMEM((BLK_ROWS, EMBED), jnp.float32),
        pltpu.SemaphoreType.DMA,
        pltpu.SemaphoreType.DMA,
        pltpu.SemaphoreType.DMA,
        pltpu.SemaphoreType.DMA,
    ],
    compiler_params=pltpu.CompilerParams(use_tc_tiling_on_sc=False),
)


@jax.jit
def kernel(tokens, word_table, pos_table):
    tok2 = tokens.astype(jnp.int32).reshape(ROWS // HALF, HALF)
    out = _grid_kernel(tok2, word_table, pos_table)
    return out.reshape(BATCH, SEQ, EMBED)
